# baseline (speedup 1.0000x reference)
import functools

import jax
import jax.numpy as jnp
import numpy as np
from jax import lax
from jax.experimental import pallas as pl
from jax.experimental.pallas import tpu as pltpu
from jax.experimental.pallas import tpu_sc as plsc

_ALPHA = 0.5
_TAU = 0.1
_K = 128
_B = 64
_N = 32768
_KPAD = 144
_NW = 32
_ROWS_PER_W = _B // _NW
_ZBUF = 4096


def _sampling_constants():
    n, k = _N, _K
    rot = [13, 15, 26, 6, 17, 29, 16, 24]
    k0, k1 = 0, 1
    ks = [np.uint32(k0), np.uint32(k1), np.uint32(k0 ^ k1 ^ 0x1BD11BDA)]
    lo = np.arange(n, dtype=np.uint32)
    x = [np.zeros(n, np.uint32) + ks[0], lo + ks[1]]

    def rotl(v, d):
        return (v << np.uint32(d)) | (v >> np.uint32(32 - d))

    with np.errstate(over="ignore"):
        for i in range(5):
            for r in rot[4 * (i % 2):4 * (i % 2) + 4]:
                x[0] = x[0] + x[1]
                x[1] = rotl(x[1], r) ^ x[0]
            x[0] = x[0] + ks[(i + 1) % 3]
            x[1] = x[1] + ks[(i + 2) % 3] + np.uint32(i + 1)
    bits = x[0] ^ x[1]

    float_bits = (bits >> np.uint32(9)) | np.uint32(0x3F800000)
    floats = float_bits.view(np.float32) - np.float32(1.0)
    tiny = np.float32(np.finfo(np.float32).tiny)
    u = np.maximum(tiny, floats * np.float32(1.0 - np.finfo(np.float32).tiny)
                   + tiny)
    gumbel = (-np.log(-np.log(u))).astype(np.float32)

    j_vals = np.arange(n, dtype=np.float32)
    log_probs = (np.float32(-(1.0 + _ALPHA - _TAU))
                 * np.log(np.float32(n) - j_vals + np.float32(1e-08)))
    m = log_probs.max()
    lse = np.float32(np.log(np.exp(log_probs - m).sum()) + m)
    probs = np.exp(log_probs - lse).astype(np.float32)

    score = gumbel + np.log(probs).astype(np.float32)
    idx = np.argsort(-score, kind="stable")[:k].astype(np.int64)

    j = idx.astype(np.float32)
    base = np.float32(n) - j + np.float32(1e-08)
    true_w = np.power(base, np.float32(-(1.0 + _ALPHA))).astype(np.float32)
    samp_p = np.power(base, np.float32(-(1.0 + _ALPHA - _TAU))).astype(
        np.float32)
    w = (true_w / (samp_p + np.float32(1e-08))).astype(np.float32)
    return idx.astype(np.int32), w


_IDX_NP, _W_NP = _sampling_constants()

_FLAT_IDX = np.zeros((_B, _KPAD), dtype=np.int32)
_FLAT_IDX[:, :_K] = (np.arange(_B, dtype=np.int64)[:, None] * _N
                     + (_N - 1 - _IDX_NP.astype(np.int64))[None, :]
                     ).astype(np.int32)
_FLAT_IDX[:, _K] = np.arange(_B, dtype=np.int32) * _N + (_N - 1)
_FLAT_IDX = _FLAT_IDX.reshape(-1)

_WCAT = np.zeros((_KPAD,), dtype=np.float32)
_WCAT[:_K] = -(_W_NP / np.float32(_K))
_WCAT[_K] = _W_NP.sum(dtype=np.float32) / np.float32(_K)


def _sc_body(xf_hbm, idx_hbm, w_hbm, out_hbm, idx_v, val_v, w_v, zb_v, fb_v,
             sem, gsem):
    wid = lax.axis_index("s") * 2 + lax.axis_index("c")
    base = wid * (_ROWS_PER_W * _KPAD)

    pltpu.sync_copy(idx_hbm.at[pl.ds(base, _ROWS_PER_W * _KPAD)], idx_v)
    pltpu.sync_copy(w_hbm, w_v)
    gather = pltpu.async_copy(xf_hbm.at[idx_v], val_v, gsem)

    zvec = jnp.zeros((16,), jnp.float32)
    for i in range(_ZBUF // 16):
        zb_v[pl.ds(16 * i, 16)] = zvec

    zero_copies = []
    for r in range(_ROWS_PER_W):
        row = _ROWS_PER_W * wid + r
        for j in range(_N // _ZBUF):
            zero_copies.append(
                pltpu.async_copy(
                    zb_v, out_hbm.at[pl.ds(row * _N + j * _ZBUF, _ZBUF)], sem))

    gather.wait()
    lane = lax.iota(jnp.int32, 16)
    for r in range(_ROWS_PER_W):
        acc = zvec
        for t in range(_KPAD // 16):
            acc = acc + (val_v[pl.ds(r * _KPAD + 16 * t, 16)]
                         * w_v[pl.ds(16 * t, 16)])
        res = acc[0]
        for i in range(1, 16):
            res = res + acc[i]
        fb_v[pl.ds(16 * r, 16)] = jnp.where(lane == 15, res, 0.0)

    for cp in zero_copies:
        cp.wait()
    fin = [
        pltpu.async_copy(
            fb_v.at[pl.ds(16 * r, 16)],
            out_hbm.at[pl.ds((_ROWS_PER_W * wid + r) * _N + _N - 16, 16)],
            sem)
        for r in range(_ROWS_PER_W)
    ]
    for cp in fin:
        cp.wait()


@functools.partial(jax.jit, static_argnums=())
def kernel(x):
    xf = x.reshape(-1)
    mesh = plsc.VectorSubcoreMesh(core_axis_name="c", subcore_axis_name="s")
    call = pl.kernel(
        _sc_body,
        out_type=jax.ShapeDtypeStruct((_B * _N,), jnp.float32),
        mesh=mesh,
        scratch_types=[
            pltpu.VMEM((_ROWS_PER_W * _KPAD,), jnp.int32),
            pltpu.VMEM((_ROWS_PER_W * _KPAD,), jnp.float32),
            pltpu.VMEM((_KPAD,), jnp.float32),
            pltpu.VMEM((_ZBUF,), jnp.float32),
            pltpu.VMEM((_ROWS_PER_W * 16,), jnp.float32),
            pltpu.SemaphoreType.DMA,
            pltpu.SemaphoreType.DMA,
        ],
    )
    out = call(xf, jnp.asarray(_FLAT_IDX), jnp.asarray(_WCAT))
    return out.reshape(_B, _N)


# speedup vs baseline: 1.6716x; 1.6716x over previous
import functools

import jax
import jax.numpy as jnp
import numpy as np
from jax import lax
from jax.experimental import pallas as pl
from jax.experimental.pallas import tpu as pltpu
from jax.experimental.pallas import tpu_sc as plsc

_ALPHA = 0.5
_TAU = 0.1
_K = 128
_B = 64
_N = 32768
_KPAD = 144
_NW = 32
_ROWS_PER_W = _B // _NW
_ZBUF = 4096


def _sampling_constants():
    n, k = _N, _K
    rot = [13, 15, 26, 6, 17, 29, 16, 24]
    k0, k1 = 0, 1
    ks = [np.uint32(k0), np.uint32(k1), np.uint32(k0 ^ k1 ^ 0x1BD11BDA)]
    lo = np.arange(n, dtype=np.uint32)
    x = [np.zeros(n, np.uint32) + ks[0], lo + ks[1]]

    def rotl(v, d):
        return (v << np.uint32(d)) | (v >> np.uint32(32 - d))

    with np.errstate(over="ignore"):
        for i in range(5):
            for r in rot[4 * (i % 2):4 * (i % 2) + 4]:
                x[0] = x[0] + x[1]
                x[1] = rotl(x[1], r) ^ x[0]
            x[0] = x[0] + ks[(i + 1) % 3]
            x[1] = x[1] + ks[(i + 2) % 3] + np.uint32(i + 1)
    bits = x[0] ^ x[1]

    float_bits = (bits >> np.uint32(9)) | np.uint32(0x3F800000)
    floats = float_bits.view(np.float32) - np.float32(1.0)
    tiny = np.float32(np.finfo(np.float32).tiny)
    u = np.maximum(tiny, floats * np.float32(1.0 - np.finfo(np.float32).tiny)
                   + tiny)
    gumbel = (-np.log(-np.log(u))).astype(np.float32)

    j_vals = np.arange(n, dtype=np.float32)
    log_probs = (np.float32(-(1.0 + _ALPHA - _TAU))
                 * np.log(np.float32(n) - j_vals + np.float32(1e-08)))
    m = log_probs.max()
    lse = np.float32(np.log(np.exp(log_probs - m).sum()) + m)
    probs = np.exp(log_probs - lse).astype(np.float32)

    score = gumbel + np.log(probs).astype(np.float32)
    idx = np.argsort(-score, kind="stable")[:k].astype(np.int64)

    j = idx.astype(np.float32)
    base = np.float32(n) - j + np.float32(1e-08)
    true_w = np.power(base, np.float32(-(1.0 + _ALPHA))).astype(np.float32)
    samp_p = np.power(base, np.float32(-(1.0 + _ALPHA - _TAU))).astype(
        np.float32)
    w = (true_w / (samp_p + np.float32(1e-08))).astype(np.float32)
    return idx.astype(np.int32), w


_IDX_NP, _W_NP = _sampling_constants()

_COLS = (_N - 1 - _IDX_NP).astype(np.int64)
_WSIGNED = -(_W_NP / np.float32(_K))
_CCUR = np.float32(_W_NP.sum(dtype=np.float32) / np.float32(_K))

_WBLK = {}
for _c, _wv in zip(_COLS.tolist(), _WSIGNED.tolist()):
    _v = _WBLK.setdefault(_c // 16, np.zeros(16, np.float32))
    _v[_c % 16] += np.float32(_wv)
_v = _WBLK.setdefault((_N - 1) // 16, np.zeros(16, np.float32))
_v[(_N - 1) % 16] += _CCUR
_BLOCKS = sorted(_WBLK)
_WTAB = np.concatenate([_WBLK[b] for b in _BLOCKS]).astype(np.float32)


def _sc_body(x_hbm, w_hbm, out_hbm, row_v, wt_v, zb_v, fb_v, sem, gsem):
    wid = lax.axis_index("s") * 2 + lax.axis_index("c")

    pltpu.sync_copy(w_hbm, wt_v)
    gathers = [
        pltpu.async_copy(x_hbm.at[_ROWS_PER_W * wid + r],
                         row_v.at[pl.ds(r * _N, _N)], gsem)
        for r in range(_ROWS_PER_W)
    ]

    zvec = jnp.zeros((16,), jnp.float32)
    for i in range(_ZBUF // 16):
        zb_v[pl.ds(16 * i, 16)] = zvec

    zero_copies = []
    for r in range(_ROWS_PER_W):
        row = _ROWS_PER_W * wid + r
        for j in range(_N // _ZBUF):
            zero_copies.append(
                pltpu.async_copy(
                    zb_v, out_hbm.at[row, pl.ds(j * _ZBUF, _ZBUF)], sem))

    for g in gathers:
        g.wait()
    lane = lax.iota(jnp.int32, 16)
    for r in range(_ROWS_PER_W):
        acc = zvec
        for t, b in enumerate(_BLOCKS):
            acc = acc + (row_v[pl.ds(r * _N + 16 * b, 16)]
                         * wt_v[pl.ds(16 * t, 16)])
        res = acc[0]
        for i in range(1, 16):
            res = res + acc[i]
        fb_v[pl.ds(16 * r, 16)] = jnp.where(lane == 15, res, 0.0)

    for cp in zero_copies:
        cp.wait()
    fin = [
        pltpu.async_copy(
            fb_v.at[pl.ds(16 * r, 16)],
            out_hbm.at[_ROWS_PER_W * wid + r, pl.ds(_N - 16, 16)],
            sem)
        for r in range(_ROWS_PER_W)
    ]
    for cp in fin:
        cp.wait()


@functools.partial(jax.jit, static_argnums=())
def kernel(x):
    mesh = plsc.VectorSubcoreMesh(core_axis_name="c", subcore_axis_name="s")
    call = pl.kernel(
        _sc_body,
        out_type=jax.ShapeDtypeStruct((_B, _N), jnp.float32),
        mesh=mesh,
        scratch_types=[
            pltpu.VMEM((_ROWS_PER_W * _N,), jnp.float32),
            pltpu.VMEM((len(_BLOCKS) * 16,), jnp.float32),
            pltpu.VMEM((_ZBUF,), jnp.float32),
            pltpu.VMEM((_ROWS_PER_W * 16,), jnp.float32),
            pltpu.SemaphoreType.DMA,
            pltpu.SemaphoreType.DMA,
        ],
    )
    return call(x, jnp.asarray(_WTAB))
